# baseline (speedup 1.0000x reference)
import functools

import jax
import jax.numpy as jnp
from jax import lax
from jax.experimental import pallas as pl
from jax.experimental.pallas import tpu as pltpu
from jax.experimental.pallas import tpu_sc as plsc

NC = 2
NS = 16
NW = NC * NS
CH = 32
UNROLL = 8
L = 16


def _sc_add_body(x_hbm, p_hbm, out_hbm, xbuf, pbuf):
    n = x_hbm.shape[0]
    S_D = p_hbm.shape[0]
    span = n // NW
    wid = lax.axis_index("s") * NC + lax.axis_index("c")
    base = wid * span
    chunk = CH * 1024
    for i in range(span // chunk):
        b0 = base + i * chunk
        tb0 = lax.rem(b0, S_D)
        pltpu.sync_copy(x_hbm.at[pl.ds(b0, chunk)], xbuf)
        pltpu.sync_copy(p_hbm.at[pl.ds(tb0, chunk)], pbuf)

        def body(j, _):
            off = j * (L * UNROLL)
            for u in range(UNROLL):
                o = off + u * L
                xbuf[pl.ds(o, L)] = xbuf[pl.ds(o, L)] + pbuf[pl.ds(o, L)]
            return 0

        lax.fori_loop(0, chunk // (L * UNROLL), body, 0)
        pltpu.sync_copy(xbuf, out_hbm.at[pl.ds(b0, chunk)])


def kernel(inputs, pos_table):
    B, S, D = inputs.shape
    x = inputs.reshape(B * S * D)
    p = pos_table.reshape(S * D)
    mesh = plsc.VectorSubcoreMesh(core_axis_name="c", subcore_axis_name="s")
    sc_add = functools.partial(
        pl.kernel,
        mesh=mesh,
        out_type=jax.ShapeDtypeStruct((B * S * D,), jnp.float32),
        scratch_types=[
            pltpu.VMEM((CH * 1024,), jnp.float32),
            pltpu.VMEM((CH * 1024,), jnp.float32),
        ],
    )(_sc_add_body)
    out = sc_add(x, p)
    return out.reshape(B, S, D)


# speedup vs baseline: 6.6677x; 6.6677x over previous
import jax
import jax.numpy as jnp
from jax.experimental import pallas as pl
from jax.experimental.pallas import tpu as pltpu

S_BLK = 2048


def _add_body(x_ref, p_ref, o_ref):
    o_ref[...] = x_ref[...] + p_ref[...]


def kernel(inputs, pos_table):
    B, S, D = inputs.shape
    grid = (S // S_BLK, B)
    return pl.pallas_call(
        _add_body,
        grid=grid,
        in_specs=[
            pl.BlockSpec((1, S_BLK, D), lambda s, b: (b, s, 0)),
            pl.BlockSpec((S_BLK, D), lambda s, b: (s, 0)),
        ],
        out_specs=pl.BlockSpec((1, S_BLK, D), lambda s, b: (b, s, 0)),
        out_shape=jax.ShapeDtypeStruct((B, S, D), inputs.dtype),
        compiler_params=pltpu.CompilerParams(
            dimension_semantics=("parallel", "parallel"),
        ),
    )(inputs, pos_table)
